# baseline (speedup 1.0000x reference)
import functools
import math

import jax
import jax.numpy as jnp
from jax import lax
from jax.experimental import pallas as pl
from jax.experimental.pallas import tpu as pltpu
from jax.experimental.pallas import tpu_sc as plsc

VOCAB = 100000
D = 128
B = 4
T = 2048
NC, NS, L = 2, 16, 16
NW = NC * NS
PW = T // NW
HW = PW // 2
HR = B * HW
SCALE = math.sqrt(D)

_mesh = plsc.VectorSubcoreMesh(core_axis_name="c", subcore_axis_name="s")


@functools.partial(
    pl.kernel,
    mesh=_mesh,
    out_type=jax.ShapeDtypeStruct((B, T, D), jnp.float32),
    scratch_types=[
        pltpu.VMEM((2, HR), jnp.int32),
        pltpu.VMEM((2 * HR, D), jnp.float32),
        pltpu.VMEM((PW, D), jnp.float32),
        pltpu.SemaphoreType.DMA,
        pltpu.SemaphoreType.DMA,
        pltpu.SemaphoreType.DMA,
        pltpu.SemaphoreType.DMA,
        pltpu.SemaphoreType.DMA,
    ],
)
def _embed(idx_hbm, tok_hbm, pos_hbm, out_hbm, idx_v, rows_v, pos_v,
           isem, psem, h0sem, h1sem, osem):
    wid = lax.axis_index("s") * NC + lax.axis_index("c")
    p0 = wid * PW

    pcopy = pltpu.async_copy(pos_hbm.at[pl.ds(p0, PW)], pos_v, psem)
    pltpu.async_copy(idx_hbm.at[wid], idx_v, isem).wait()
    hsems = (h0sem, h1sem)
    gathers = [
        pltpu.async_copy(
            tok_hbm.at[idx_v.at[h]],
            rows_v.at[pl.ds(h * HR, HR)], hsems[h])
        for h in range(2)
    ]

    out_waits = []
    for h, g in enumerate(gathers):
        g.wait()
        if h == 0:
            pcopy.wait()

        def body(i, carry, h=h):
            pi = h * HW + i
            for j in range(D // L):
                sl = pl.ds(j * L, L)
                pv = pos_v[pi, sl]
                for b in range(B):
                    row = h * HR + b * HW + i
                    rows_v[row, sl] = rows_v[row, sl] * SCALE + pv
            return carry

        lax.fori_loop(0, HW, body, 0)
        for b in range(B):
            out_waits.append(pltpu.async_copy(
                rows_v.at[pl.ds(h * HR + b * HW, HW)],
                out_hbm.at[b, pl.ds(p0 + h * HW, HW)], osem))

    for wt in out_waits:
        wt.wait()


def kernel(token_ids, tok_table, pos_table):
    idx = (token_ids.astype(jnp.int32)
           .reshape(B, NW, 2, HW)
           .transpose(1, 2, 0, 3)
           .reshape(NW, 2, HR))
    out = _embed(idx, tok_table, pos_table)
    return out
